# baseline (speedup 1.0000x reference)
import functools

import jax
import jax.numpy as jnp
import numpy as np
from jax import lax
from jax.experimental import pallas as pl
from jax.experimental.pallas import tpu as pltpu
from jax.experimental.pallas import tpu_sc as plsc

_BASE = [2654435761, 2246822519, 3266489917, 2028178513, 1220703125,
         1610612741, 805306457, 402653189, 3674653429, 2860486313,
         1073676287, 2971215073, 1500450271, 3267000013, 2654435789,
         4049292737]
_EXTRA = _BASE + [2246822531, 3266489927, 2028178519, 1220703133, 1610612759,
                  805306463, 402653201, 3674653441, 2860486319, 1073676311,
                  2971215091, 1500450277, 3267000023, 2654435801, 4049292751,
                  2246822537]
_COND_PRIMES = _BASE[:8]
_NS = 8
_BUCKETS = 65536
_EMB = 16
_MASK16 = 0xFFFF
_Z = np.int32(0)

_M5 = [((p ^ 3735928559) & _MASK16) for p in _EXTRA]
_M6 = [((p ^ 3405691582) & _MASK16) for p in _BASE[:8]]
_M7 = [((p ^ 2343432205) & _MASK16) for p in _BASE[:8]]
_MBASE = [p & _MASK16 for p in _BASE]
_CPL = [p & _MASK16 for p in _COND_PRIMES]


def _hash_body(tok_ref, keys_ref):
    tok = tok_ref[...]
    bb, t = tok.shape
    pad = jnp.zeros((bb, 128), jnp.int32)
    tokp = jnp.concatenate([pad, tok], axis=1)

    def shift(o):
        return lax.slice(tokp, (0, 128 - o), (bb, 128 - o + t))

    acc = None
    bounds = [1, 2, 4, 8, 16]
    ks = []
    o = 1
    for s, w in enumerate(bounds):
        while o <= w:
            term = shift(o) * np.int32(_MBASE[o - 1])
            acc = term if acc is None else acc ^ term
            o += 1
        ks.append(acc)

    acc5 = shift(1) * np.int32(_M5[0])
    for i in range(1, 32):
        acc5 = acc5 ^ (shift(i + 1) * np.int32(_M5[i]))
    ks.append(acc5)

    acc6 = None
    for i in range(64):
        term = shift(i + 1) * np.int32(_M6[i % 8])
        acc6 = term if acc6 is None else acc6 ^ term
    ks.append(acc6)

    acc7 = None
    for i in range(128):
        term = shift(i + 1) * np.int32(_M7[i % 8])
        acc7 = term if acc7 is None else acc7 ^ term
    ks.append(acc7)

    for s in range(_NS):
        k = ks[s] & np.int32(_MASK16)
        if s < 4:
            k = k + np.int32(s * _BUCKETS)
        keys_ref[s] = k


def _hash_keys(tok):
    b, t = tok.shape
    bb = 16
    grid = b // bb
    return pl.pallas_call(
        _hash_body,
        grid=(grid,),
        in_specs=[pl.BlockSpec((bb, t), lambda i: (i, _Z))],
        out_specs=pl.BlockSpec((_NS, bb, t), lambda i: (_Z, i, _Z)),
        out_shape=jax.ShapeDtypeStruct((_NS, b, t), jnp.int32),
    )(tok)


def _cond_body(rows_ref, w_ref, keys_ref, idx_ref):
    rows = rows_ref[...]
    logits_t = lax.dot_general(
        w_ref[...], rows, (((1,), (1,)), ((), ())),
        preferred_element_type=jnp.float32,
        precision=lax.Precision.HIGHEST)
    ck = None
    for i in range(8):
        term = jnp.where(logits_t[i:i + 1] > 0,
                         np.int32(_CPL[i]), np.int32(0))
        ck = term if ck is None else ck ^ term
    keys = keys_ref[...]
    nb = keys.shape[1]
    offs = (lax.broadcasted_iota(jnp.int32, (4, nb), 0)
            + np.int32(4)) * np.int32(_BUCKETS)
    idx_ref[...] = ((keys ^ ck) & np.int32(_MASK16)) + offs


def _cond_indices(short_rows, w, keys_long):
    bt = short_rows.shape[0]
    nb = 2048
    grid = bt // nb
    return pl.pallas_call(
        _cond_body,
        grid=(grid,),
        in_specs=[
            pl.BlockSpec((nb, 64), lambda i: (i, _Z)),
            pl.BlockSpec((8, 64), lambda i: (_Z, _Z)),
            pl.BlockSpec((4, nb), lambda i: (_Z, i)),
        ],
        out_specs=pl.BlockSpec((4, nb), lambda i: (_Z, i)),
        out_shape=jax.ShapeDtypeStruct((4, bt), jnp.int32),
    )(short_rows, w, keys_long)


try:
    _INFO = plsc.get_sparse_core_info()
    _NC = _INFO.num_cores
    _NSUB = _INFO.num_subcores
except Exception:
    _NC, _NSUB = 2, 16
_NW = _NC * _NSUB


def _make_short_gather(bt):
    tpw = bt // _NW
    mesh = plsc.VectorSubcoreMesh(core_axis_name="c", subcore_axis_name="s",
                                  num_cores=_NC)

    @functools.partial(
        pl.kernel, mesh=mesh,
        out_type=jax.ShapeDtypeStruct((bt, 4 * _EMB), jnp.float32),
        compiler_params=pltpu.CompilerParams(use_tc_tiling_on_sc=False),
        scratch_types=[
            pltpu.VMEM((tpw,), jnp.int32),
            pltpu.VMEM((tpw, _EMB), jnp.float32),
            pltpu.SemaphoreType.DMA,
        ],
    )
    def k2(tables_hbm, idx_hbm, out_hbm, idx_v, rows_v, sem):
        wid = (lax.axis_index("s").astype(jnp.int32) * np.int32(_NC)
               + lax.axis_index("c").astype(jnp.int32))
        base = wid * np.int32(tpw)
        for s in range(4):
            pltpu.sync_copy(idx_hbm.at[np.int32(s), pl.ds(base, tpw)], idx_v)
            pltpu.async_copy(tables_hbm.at[idx_v], rows_v, sem).wait()
            pltpu.sync_copy(
                rows_v,
                out_hbm.at[pl.ds(base, tpw),
                           pl.ds(np.int32(16 * s), 16)])

    return k2


def _make_long_gather(bt):
    tpw = bt // _NW
    mesh = plsc.VectorSubcoreMesh(core_axis_name="c", subcore_axis_name="s",
                                  num_cores=_NC)

    @functools.partial(
        pl.kernel, mesh=mesh,
        out_type=jax.ShapeDtypeStruct((bt, _NS * _EMB), jnp.float32),
        compiler_params=pltpu.CompilerParams(use_tc_tiling_on_sc=False),
        scratch_types=[
            pltpu.VMEM((tpw,), jnp.int32),
            pltpu.VMEM((tpw, _EMB), jnp.float32),
            pltpu.SemaphoreType.DMA,
        ],
    )
    def k4(tables_hbm, idx_hbm, short_hbm, out_hbm, idx_v, rows_v, sem):
        wid = (lax.axis_index("s").astype(jnp.int32) * np.int32(_NC)
               + lax.axis_index("c").astype(jnp.int32))
        base = wid * np.int32(tpw)
        pltpu.sync_copy(
            short_hbm.at[pl.ds(base, tpw)],
            out_hbm.at[pl.ds(base, tpw), pl.ds(np.int32(0), 64)])
        for s in range(4):
            pltpu.sync_copy(idx_hbm.at[np.int32(s), pl.ds(base, tpw)], idx_v)
            pltpu.async_copy(tables_hbm.at[idx_v], rows_v, sem).wait()
            pltpu.sync_copy(
                rows_v,
                out_hbm.at[pl.ds(base, tpw),
                           pl.ds(np.int32(64 + 16 * s), 16)])

    return k4


def kernel(tokens, tables, cond_proj_w):
    b, t = tokens.shape
    bt = b * t
    tok = tokens.astype(jnp.int32)
    tables_flat = tables.reshape(_NS * _BUCKETS, _EMB).astype(jnp.float32)
    w = cond_proj_w.astype(jnp.float32)

    keys = _hash_keys(tok).reshape(_NS, bt)
    idx_short = keys[:4]
    keys_long = keys[4:]

    short_cat = _make_short_gather(bt)(tables_flat, idx_short)
    idx_long = _cond_indices(short_cat, w, keys_long)
    out = _make_long_gather(bt)(tables_flat, idx_long, short_cat)
    return out.reshape(b, t, _NS, _EMB)


# speedup vs baseline: 1.5381x; 1.5381x over previous
import functools

import jax
import jax.numpy as jnp
import numpy as np
from jax import lax
from jax.experimental import pallas as pl
from jax.experimental.pallas import tpu as pltpu
from jax.experimental.pallas import tpu_sc as plsc

_BASE = [2654435761, 2246822519, 3266489917, 2028178513, 1220703125,
         1610612741, 805306457, 402653189, 3674653429, 2860486313,
         1073676287, 2971215073, 1500450271, 3267000013, 2654435789,
         4049292737]
_EXTRA = _BASE + [2246822531, 3266489927, 2028178519, 1220703133, 1610612759,
                  805306463, 402653201, 3674653441, 2860486319, 1073676311,
                  2971215091, 1500450277, 3267000023, 2654435801, 4049292751,
                  2246822537]
_COND_PRIMES = _BASE[:8]
_NS = 8
_BUCKETS = 65536
_EMB = 16
_MASK16 = 0xFFFF
_Z = np.int32(0)

_M5 = [((p ^ 3735928559) & _MASK16) for p in _EXTRA]
_M6 = [((p ^ 3405691582) & _MASK16) for p in _BASE[:8]]
_M7 = [((p ^ 2343432205) & _MASK16) for p in _BASE[:8]]
_MBASE = [p & _MASK16 for p in _BASE]
_CPL = [p & _MASK16 for p in _COND_PRIMES]


def _hash_body(tok_ref, keys_ref):
    tok = tok_ref[...]
    bb, t = tok.shape
    pad = jnp.zeros((bb, 128), jnp.int32)
    tokp = jnp.concatenate([pad, tok], axis=1)

    def shift(o):
        return lax.slice(tokp, (0, 128 - o), (bb, 128 - o + t))

    acc = None
    bounds = [1, 2, 4, 8, 16]
    ks = []
    o = 1
    for s, w in enumerate(bounds):
        while o <= w:
            term = shift(o) * np.int32(_MBASE[o - 1])
            acc = term if acc is None else acc ^ term
            o += 1
        ks.append(acc)

    acc5 = shift(1) * np.int32(_M5[0])
    for i in range(1, 32):
        acc5 = acc5 ^ (shift(i + 1) * np.int32(_M5[i]))
    ks.append(acc5)

    acc6 = None
    for i in range(64):
        term = shift(i + 1) * np.int32(_M6[i % 8])
        acc6 = term if acc6 is None else acc6 ^ term
    ks.append(acc6)

    acc7 = None
    for i in range(128):
        term = shift(i + 1) * np.int32(_M7[i % 8])
        acc7 = term if acc7 is None else acc7 ^ term
    ks.append(acc7)

    for s in range(_NS):
        k = ks[s] & np.int32(_MASK16)
        if s < 4:
            k = k + np.int32(s * _BUCKETS)
        keys_ref[s] = k


def _hash_keys(tok):
    b, t = tok.shape
    bb = 16
    grid = b // bb
    return pl.pallas_call(
        _hash_body,
        grid=(grid,),
        in_specs=[pl.BlockSpec((bb, t), lambda i: (i, _Z))],
        out_specs=pl.BlockSpec((_NS, bb, t), lambda i: (_Z, i, _Z)),
        out_shape=jax.ShapeDtypeStruct((_NS, b, t), jnp.int32),
    )(tok)


def _cond_body(rows_ref, w_ref, keys_ref, idx_ref):
    rows = rows_ref[...]
    logits_t = lax.dot_general(
        w_ref[...], rows, (((1,), (1,)), ((), ())),
        preferred_element_type=jnp.float32,
        precision=lax.Precision.HIGHEST)
    ck = None
    for i in range(8):
        term = jnp.where(logits_t[i:i + 1] > 0,
                         np.int32(_CPL[i]), np.int32(0))
        ck = term if ck is None else ck ^ term
    keys = keys_ref[...]
    nb = keys.shape[1]
    offs = (lax.broadcasted_iota(jnp.int32, (4, nb), 0)
            + np.int32(4)) * np.int32(_BUCKETS)
    idx_ref[...] = ((keys ^ ck) & np.int32(_MASK16)) + offs


def _cond_indices(short_rows, w, keys_long):
    bt = short_rows.shape[0]
    nb = 2048
    grid = bt // nb
    return pl.pallas_call(
        _cond_body,
        grid=(grid,),
        in_specs=[
            pl.BlockSpec((nb, 64), lambda i: (i, _Z)),
            pl.BlockSpec((8, 64), lambda i: (_Z, _Z)),
            pl.BlockSpec((4, nb), lambda i: (_Z, i)),
        ],
        out_specs=pl.BlockSpec((4, nb), lambda i: (_Z, i)),
        out_shape=jax.ShapeDtypeStruct((4, bt), jnp.int32),
    )(short_rows, w, keys_long)


try:
    _INFO = plsc.get_sparse_core_info()
    _NC = _INFO.num_cores
    _NSUB = _INFO.num_subcores
except Exception:
    _NC, _NSUB = 2, 16
_NW = _NC * _NSUB

_CN = 1024
_RING = 6


def _make_gather(n_rows):
    rpw = n_rows // _NW
    nch = rpw // _CN
    mesh = plsc.VectorSubcoreMesh(core_axis_name="c", subcore_axis_name="s",
                                  num_cores=_NC)
    r = min(_RING, nch)

    scratch = ([pltpu.VMEM((_CN,), jnp.int32) for _ in range(r)]
               + [pltpu.VMEM((_CN, _EMB), jnp.float32) for _ in range(r)]
               + [pltpu.SemaphoreType.DMA for _ in range(2 * r)])

    @functools.partial(
        pl.kernel, mesh=mesh,
        out_type=jax.ShapeDtypeStruct((n_rows, _EMB), jnp.float32),
        compiler_params=pltpu.CompilerParams(use_tc_tiling_on_sc=False),
        scratch_types=scratch,
    )
    def k(tables_hbm, idx_hbm, out_hbm, *bufs):
        idxs = bufs[:r]
        rows = bufs[r:2 * r]
        gsems = bufs[2 * r:3 * r]
        wsems = bufs[3 * r:4 * r]
        wid = (lax.axis_index("s").astype(jnp.int32) * np.int32(_NC)
               + lax.axis_index("c").astype(jnp.int32))
        base = wid * np.int32(rpw)

        ghandles = [None] * nch
        whandles = [None] * nch

        def issue(i):
            b = i % r
            off = base + np.int32(i * _CN)
            pltpu.sync_copy(idx_hbm.at[pl.ds(off, _CN)], idxs[b])
            ghandles[i] = pltpu.async_copy(tables_hbm.at[idxs[b]], rows[b],
                                           gsems[b])

        for i in range(r):
            issue(i)
        for i in range(nch):
            b = i % r
            ghandles[i].wait()
            off = base + np.int32(i * _CN)
            whandles[i] = pltpu.async_copy(rows[b],
                                           out_hbm.at[pl.ds(off, _CN)],
                                           wsems[b])
            j = i + r
            if j < nch:
                whandles[i].wait()
                issue(j)
        for i in range(max(0, nch - r), nch):
            if whandles[i] is not None and i + r >= nch:
                whandles[i].wait()

    return k


def kernel(tokens, tables, cond_proj_w):
    b, t = tokens.shape
    bt = b * t
    tok = tokens.astype(jnp.int32)
    tables_flat = tables.reshape(_NS * _BUCKETS, _EMB).astype(jnp.float32)
    w = cond_proj_w.astype(jnp.float32)

    keys = _hash_keys(tok).reshape(_NS, bt)
    keys_short = keys[:4]
    keys_long = keys[4:]

    idx_short_il = keys_short.T.reshape(4 * bt)
    short_rows = _make_gather(4 * bt)(tables_flat, idx_short_il)
    short_cat = short_rows.reshape(bt, 4 * _EMB)

    idx_long = _cond_indices(short_cat, w, keys_long)

    idx_full_il = jnp.concatenate([keys_short, idx_long], axis=0
                                  ).T.reshape(_NS * bt)
    out = _make_gather(_NS * bt)(tables_flat, idx_full_il)
    return out.reshape(b, t, _NS, _EMB)
